# baseline (speedup 1.0000x reference)
import jax
import jax.numpy as jnp
from jax.experimental import pallas as pl
from jax.experimental.pallas import tpu as pltpu

IN_FEATURES = 16
OUT_FEATURES = 32
PACK = 8
N_PACKED = PACK * OUT_FEATURES
ROW_TILE = 8192
NUM_CORES = 2


def _make_body(n_blocks, tb, half, n_rows):
    tb8 = tb // PACK

    def body(x_hbm, w_ref, b_ref, other_hbm, out_hbm,
             x_buf, o_buf, y_buf, sx, so, sy):
        p = pl.program_id(0)
        base = p * half

        xv = x_hbm.reshape(n_rows // PACK, PACK, IN_FEATURES)
        ov = other_hbm.reshape(n_rows // PACK, PACK, OUT_FEATURES)

        def in_copies(i, slot):
            t0 = pl.multiple_of((base + i * tb) // PACK, 8)
            cs = []
            for s in range(PACK):
                cs.append(pltpu.make_async_copy(
                    xv.at[pl.ds(t0, tb8), s, :],
                    x_buf.at[slot, s], sx.at[slot]))
                cs.append(pltpu.make_async_copy(
                    ov.at[pl.ds(t0, tb8), s, :],
                    o_buf.at[slot, s], so.at[slot]))
            return cs

        def out_copy(i, slot):
            t0 = pl.multiple_of((base + i * tb) // PACK, 8)
            return pltpu.make_async_copy(y_buf.at[slot],
                                         out_hbm.at[pl.ds(t0, tb8), :],
                                         sy.at[slot])

        w = w_ref[:, :OUT_FEATURES]
        b = b_ref[:, :OUT_FEATURES]

        for c in in_copies(0, 0):
            c.start()
        for i in range(n_blocks):
            slot = i % 2
            if i + 1 < n_blocks:
                for c in in_copies(i + 1, 1 - slot):
                    c.start()
            for c in in_copies(i, slot):
                c.wait()
            if i >= 2:
                out_copy(i - 2, slot).wait()
            for s in range(PACK):
                xs = x_buf[slot, s]
                os_ = o_buf[slot, s]
                v = jnp.dot(xs, w, preferred_element_type=jnp.float32)
                y_buf[slot, :, s * OUT_FEATURES:(s + 1) * OUT_FEATURES] = (
                    jnp.maximum(v + b + os_, 0.0))
            out_copy(i, slot).start()
        for k in range(max(n_blocks - 2, 0), n_blocks):
            out_copy(k, k % 2).wait()

    return body


@jax.jit
def kernel(x, w_padded, b_padded, other):
    B = x.shape[0]
    half = B // NUM_CORES
    tb = min(ROW_TILE, half)
    while half % tb or tb % PACK:
        tb -= 1
    n_blocks = half // tb

    out_packed = pl.pallas_call(
        _make_body(n_blocks, tb, half, B),
        out_shape=jax.ShapeDtypeStruct((B // PACK, N_PACKED), jnp.float32),
        grid=(NUM_CORES,),
        in_specs=[
            pl.BlockSpec(memory_space=pl.ANY),
            pl.BlockSpec((IN_FEATURES, 128), lambda i: (0, 0)),
            pl.BlockSpec((1, 128), lambda i: (0, 0)),
            pl.BlockSpec(memory_space=pl.ANY),
        ],
        out_specs=pl.BlockSpec(memory_space=pl.ANY),
        scratch_shapes=[
            pltpu.VMEM((2, PACK, tb // PACK, IN_FEATURES), jnp.float32),
            pltpu.VMEM((2, PACK, tb // PACK, OUT_FEATURES), jnp.float32),
            pltpu.VMEM((2, tb // PACK, N_PACKED), jnp.float32),
            pltpu.SemaphoreType.DMA((2,)),
            pltpu.SemaphoreType.DMA((2,)),
            pltpu.SemaphoreType.DMA((2,)),
        ],
        compiler_params=pltpu.CompilerParams(
            dimension_semantics=("parallel",),
        ),
    )(x, w_padded, b_padded, other)

    return out_packed.reshape(B, OUT_FEATURES)


# speedup vs baseline: 1.1552x; 1.1552x over previous
import jax
import jax.numpy as jnp
from jax.experimental import pallas as pl
from jax.experimental.pallas import tpu as pltpu

IN_FEATURES = 16
OUT_FEATURES = 32
OUT_WIDE = 128
ROW_TILE = 8192
NUM_CORES = 2


def _make_body(n_blocks, tb, half):
    def body(x_hbm, w_ref, b_ref, other_hbm, out_hbm,
             x_buf, o_buf, y_buf, sx, so, sy):
        p = pl.program_id(0)
        base = p * half

        def in_copies(i, slot):
            r0 = base + i * tb
            return (
                pltpu.make_async_copy(x_hbm.at[pl.ds(r0, tb), :],
                                      x_buf.at[slot], sx.at[slot]),
                pltpu.make_async_copy(other_hbm.at[pl.ds(r0, tb), :],
                                      o_buf.at[slot], so.at[slot]),
            )

        def out_copy(i, slot):
            r0 = base + i * tb
            return pltpu.make_async_copy(y_buf.at[slot],
                                         out_hbm.at[pl.ds(r0, tb), :],
                                         sy.at[slot])

        for c in in_copies(0, 0):
            c.start()
        for i in range(n_blocks):
            slot = i % 2
            if i + 1 < n_blocks:
                for c in in_copies(i + 1, 1 - slot):
                    c.start()
            for c in in_copies(i, slot):
                c.wait()
            if i >= 2:
                out_copy(i - 2, slot).wait()
            v = jnp.dot(x_buf[slot], w_ref[...],
                        preferred_element_type=jnp.float32)
            o128 = jnp.pad(o_buf[slot][...],
                           ((0, 0), (0, OUT_WIDE - OUT_FEATURES)))
            y_buf[slot] = jnp.maximum(v + b_ref[...] + o128, 0.0)
            out_copy(i, slot).start()
        for k in range(max(n_blocks - 2, 0), n_blocks):
            out_copy(k, k % 2).wait()

    return body


@jax.jit
def kernel(x, w_padded, b_padded, other):
    B = x.shape[0]
    half = B // NUM_CORES
    tb = min(ROW_TILE, half)
    while half % tb:
        tb -= 1
    n_blocks = half // tb

    out_wide = pl.pallas_call(
        _make_body(n_blocks, tb, half),
        out_shape=jax.ShapeDtypeStruct((B, OUT_WIDE), jnp.float32),
        grid=(NUM_CORES,),
        in_specs=[
            pl.BlockSpec(memory_space=pl.ANY),
            pl.BlockSpec((IN_FEATURES, OUT_WIDE), lambda i: (0, 0)),
            pl.BlockSpec((1, OUT_WIDE), lambda i: (0, 0)),
            pl.BlockSpec(memory_space=pl.ANY),
        ],
        out_specs=pl.BlockSpec(memory_space=pl.ANY),
        scratch_shapes=[
            pltpu.VMEM((2, tb, IN_FEATURES), jnp.float32),
            pltpu.VMEM((2, tb, OUT_FEATURES), jnp.float32),
            pltpu.VMEM((2, tb, OUT_WIDE), jnp.float32),
            pltpu.SemaphoreType.DMA((2,)),
            pltpu.SemaphoreType.DMA((2,)),
            pltpu.SemaphoreType.DMA((2,)),
        ],
        compiler_params=pltpu.CompilerParams(
            dimension_semantics=("parallel",),
        ),
    )(x, w_padded, b_padded, other)

    return out_wide[:, :OUT_FEATURES]
